# baseline (speedup 1.0000x reference)
import functools

import jax
import jax.numpy as jnp
from jax.experimental import pallas as pl
from jax.experimental.pallas import tpu as pltpu

KEEP_THRESHOLD = 0.01
NEG = -10000.0


def _tc_body(mask_ref, probs_ref, out_ref, acc_ref, *, nblk, rows):
    c = pl.program_id(1)

    tile = probs_ref[0, 0, :, :]
    m = mask_ref[0, :, :]
    masked = jnp.where(m < 0.0, 0.0, tile)
    partial = jnp.sum(masked, axis=0, keepdims=True)

    @pl.when(c == 0)
    def _init():
        acc_ref[...] = partial

    @pl.when(c != 0)
    def _accum():
        acc_ref[...] += partial

    @pl.when(c == pl.num_programs(1) - 1)
    def _epilogue():
        scores = acc_ref[...]
        mx = jnp.max(scores)
        rel = scores / mx
        out_ref[0, 0, :, :] = jnp.where(rel < KEEP_THRESHOLD, NEG, 0.0)


def kernel(attention_mask, attention_probs, sentence_lengths):
    del sentence_lengths
    B, H, S, _ = attention_probs.shape
    rows = 512
    nblk = S // rows

    mask3 = attention_mask.reshape(B, S, 1)

    out = pl.pallas_call(
        functools.partial(_tc_body, nblk=nblk, rows=rows),
        grid=(B, H * nblk),
        in_specs=[
            pl.BlockSpec((1, rows, 1), lambda b, c: (b, c % nblk, 0)),
            pl.BlockSpec((1, 1, rows, S), lambda b, c: (b, c // nblk, c % nblk, 0)),
        ],
        out_specs=pl.BlockSpec((1, 1, 1, S), lambda b, c: (b, 0, 0, 0)),
        out_shape=jax.ShapeDtypeStruct((B, 1, 1, S), jnp.float32),
        scratch_shapes=[pltpu.VMEM((1, S), jnp.float32)],
    )(mask3, attention_probs)
    return out
